# Initial kernel scaffold; baseline (speedup 1.0000x reference)
import jax
import jax.numpy as jnp
from jax.experimental import pallas as pl


def kernel(xyz, points, W0, b0, g0, be0, W1, b1, g1, be1, W2, b2, g2, be2):
    raise NotImplementedError("write your pallas kernel here")


# speedup vs baseline: 13.8206x; 13.8206x over previous
import functools

import jax
import jax.numpy as jnp
from jax import lax
from jax.experimental import pallas as pl
from jax.experimental.pallas import tpu as pltpu
from jax.experimental.pallas import tpu_sc as plsc

B = 8
N = 4096
D = 64
S = 512
K = 32
RADIUS = 0.2
CPAD = 128
ROWS = B * S * K
GROUPS = B * S


def _fps_body(xyz_ref, far0_ref, nxt_ref):
    x = xyz_ref[:, 0, :]
    y = xyz_ref[:, 1, :]
    z = xyz_ref[:, 2, :]
    iota = lax.broadcasted_iota(jnp.int32, (B, N), 1)

    def body(i, carry):
        dist, far = carry
        m = iota == far
        cx = jnp.sum(jnp.where(m, x, 0.0), axis=1, keepdims=True)
        cy = jnp.sum(jnp.where(m, y, 0.0), axis=1, keepdims=True)
        cz = jnp.sum(jnp.where(m, z, 0.0), axis=1, keepdims=True)
        nxt_ref[pl.ds(i, 1), :, :] = jnp.concatenate([cx, cy, cz], axis=1).reshape(1, B, 3)
        d = (x - cx) ** 2 + (y - cy) ** 2 + (z - cz) ** 2
        dist = jnp.minimum(dist, d)
        mx = jnp.max(dist, axis=1, keepdims=True)
        far_new = jnp.min(jnp.where(dist == mx, iota, N), axis=1, keepdims=True)
        return dist, far_new

    dist0 = jnp.full((B, N), 1e10, dtype=jnp.float32)
    lax.fori_loop(0, S, body, (dist0, far0_ref[...]))


def _fps(xyz, far0):
    return pl.pallas_call(
        _fps_body,
        out_shape=jax.ShapeDtypeStruct((S, B, 3), jnp.float32),
    )(xyz, far0)


SBLK = 256


def _ballq_body(nxt_ref, xyz_ref, idx_ref):
    b = pl.program_id(0)
    nxt = nxt_ref[0]
    xb = xyz_ref[0]
    s2 = jnp.sum(nxt * nxt, axis=1, keepdims=True)
    d2 = jnp.sum(xb * xb, axis=0, keepdims=True)
    prod = lax.dot_general(nxt, xb, (((1,), (0,)), ((), ())),
                           preferred_element_type=jnp.float32)
    dist = s2 + d2 - 2.0 * prod
    maskf = (dist <= RADIUS * RADIUS).astype(jnp.float32)

    m2 = maskf.reshape(SBLK * 32, 128)
    r = lax.broadcasted_iota(jnp.int32, (128, 128), 0)
    c = lax.broadcasted_iota(jnp.int32, (128, 128), 1)
    tri_incl = (r <= c).astype(jnp.float32)
    cs_in = lax.dot_general(m2, tri_incl, (((1,), (0,)), ((), ())),
                            preferred_element_type=jnp.float32)
    cs_in = cs_in.reshape(SBLK, 32, 128)
    tot = jnp.sum(maskf.reshape(SBLK, 32, 128), axis=2)
    r32 = lax.broadcasted_iota(jnp.int32, (32, 32), 0)
    c32 = lax.broadcasted_iota(jnp.int32, (32, 32), 1)
    tri_excl = (r32 < c32).astype(jnp.float32)
    off = lax.dot_general(tot, tri_excl, (((1,), (0,)), ((), ())),
                          preferred_element_type=jnp.float32)
    csum = cs_in + off[:, :, None]
    cnt = jnp.sum(tot, axis=1, keepdims=True)

    base = (b * N).astype(jnp.int32) if hasattr(b, "astype") else b * N
    raw0 = jnp.sum((csum <= 0.0).astype(jnp.float32), axis=(1, 2))[:, None]
    first = jnp.where(cnt > 0.0, raw0, 0.0)
    for k in range(K):
        if k == 0:
            raw = raw0
        else:
            raw = jnp.sum((csum <= float(k)).astype(jnp.float32), axis=(1, 2))[:, None]
        sel = jnp.where(float(k) < cnt, raw, first)
        idx_ref[0, :, pl.ds(k, 1)] = sel.astype(jnp.int32) + base


def _ballq(nxt, xyz):
    grid = (B, S // SBLK)
    return pl.pallas_call(
        _ballq_body,
        grid=grid,
        in_specs=[
            pl.BlockSpec((1, SBLK, 3), lambda b, s: (b, s, 0)),
            pl.BlockSpec((1, 3, N), lambda b, s: (b, 0, 0)),
        ],
        out_specs=pl.BlockSpec((1, SBLK, K), lambda b, s: (b, s, 0)),
        out_shape=jax.ShapeDtypeStruct((B, S, K), jnp.int32),
    )(nxt, xyz)


GCHUNK = 128
PER_W = ROWS // 32


def _sc_gather(table, idx_flat):
    mesh = plsc.VectorSubcoreMesh(core_axis_name="c", subcore_axis_name="s")

    @functools.partial(
        pl.kernel,
        mesh=mesh,
        out_type=jax.ShapeDtypeStruct((ROWS, CPAD), jnp.float32),
        scratch_types=[
            pltpu.VMEM((GCHUNK,), jnp.int32),
            pltpu.VMEM((GCHUNK, CPAD), jnp.float32),
            pltpu.SemaphoreType.DMA,
        ],
    )
    def gather_k(table_hbm, idx_hbm, out_hbm, idx_v, rows_v, sem):
        wid = lax.axis_index("s") * 2 + lax.axis_index("c")
        base = wid * PER_W

        def body(i, carry):
            off = base + i * GCHUNK
            pltpu.sync_copy(idx_hbm.at[pl.ds(off, GCHUNK)], idx_v)
            pltpu.async_copy(table_hbm.at[idx_v], rows_v, sem).wait()
            pltpu.sync_copy(rows_v, out_hbm.at[pl.ds(off, GCHUNK)])
            return carry

        lax.fori_loop(0, PER_W // GCHUNK, body, 0)

    return gather_k(table, idx_flat)


RBLK = 8192
GBLK = RBLK // K


def _mlp1_body(xg_ref, nxt_ref, w_ref, b_ref, y_ref, st_ref):
    i = pl.program_id(0)
    xg = xg_ref[...]
    w = w_ref[...]
    nxt = nxt_ref[...]
    c1 = lax.dot_general(nxt, w[0:3, :], (((1,), (0,)), ((), ())),
                         preferred_element_type=jnp.float32)
    c1r = jnp.broadcast_to(c1[:, None, :], (GBLK, K, 64)).reshape(RBLK, 64)
    y = lax.dot_general(xg, w, (((1,), (0,)), ((), ())),
                        preferred_element_type=jnp.float32) + b_ref[...] - c1r
    y_ref[...] = y

    @pl.when(i == 0)
    def _():
        st_ref[...] = jnp.zeros_like(st_ref)

    st_ref[0:1, :] += jnp.sum(y, axis=0, keepdims=True)
    st_ref[1:2, :] += jnp.sum(y * y, axis=0, keepdims=True)


def _mlp1(xg, nxt_rows, w0p, b0):
    grid = (ROWS // RBLK,)
    return pl.pallas_call(
        _mlp1_body,
        grid=grid,
        in_specs=[
            pl.BlockSpec((RBLK, CPAD), lambda i: (i, 0)),
            pl.BlockSpec((GBLK, 3), lambda i: (i, 0)),
            pl.BlockSpec((CPAD, 64), lambda i: (0, 0)),
            pl.BlockSpec((1, 64), lambda i: (0, 0)),
        ],
        out_specs=[
            pl.BlockSpec((RBLK, 64), lambda i: (i, 0)),
            pl.BlockSpec((8, 64), lambda i: (0, 0)),
        ],
        out_shape=[
            jax.ShapeDtypeStruct((ROWS, 64), jnp.float32),
            jax.ShapeDtypeStruct((8, 64), jnp.float32),
        ],
    )(xg, nxt_rows, w0p, b0)


def _mlp_body(y_ref, a_ref, d_ref, w_ref, b_ref, o_ref, st_ref):
    i = pl.program_id(0)
    h = jnp.maximum(y_ref[...] * a_ref[...] + d_ref[...], 0.0)
    y = lax.dot_general(h, w_ref[...], (((1,), (0,)), ((), ())),
                        preferred_element_type=jnp.float32) + b_ref[...]
    o_ref[...] = y

    @pl.when(i == 0)
    def _():
        st_ref[...] = jnp.zeros_like(st_ref)

    st_ref[0:1, :] += jnp.sum(y, axis=0, keepdims=True)
    st_ref[1:2, :] += jnp.sum(y * y, axis=0, keepdims=True)


def _mlp_layer(y, a, d, wt, b):
    cin = y.shape[1]
    cout = wt.shape[1]
    grid = (ROWS // RBLK,)
    return pl.pallas_call(
        _mlp_body,
        grid=grid,
        in_specs=[
            pl.BlockSpec((RBLK, cin), lambda i: (i, 0)),
            pl.BlockSpec((1, cin), lambda i: (0, 0)),
            pl.BlockSpec((1, cin), lambda i: (0, 0)),
            pl.BlockSpec((cin, cout), lambda i: (0, 0)),
            pl.BlockSpec((1, cout), lambda i: (0, 0)),
        ],
        out_specs=[
            pl.BlockSpec((RBLK, cout), lambda i: (i, 0)),
            pl.BlockSpec((8, cout), lambda i: (0, 0)),
        ],
        out_shape=[
            jax.ShapeDtypeStruct((ROWS, cout), jnp.float32),
            jax.ShapeDtypeStruct((8, cout), jnp.float32),
        ],
    )(y, a, d, wt, b)


def _pool_body(y_ref, a_ref, d_ref, o_ref):
    h = jnp.maximum(y_ref[...] * a_ref[...] + d_ref[...], 0.0)
    o_ref[...] = jnp.max(h.reshape(GBLK, K, 128), axis=1)


def _pool(y, a, d):
    grid = (ROWS // RBLK,)
    return pl.pallas_call(
        _pool_body,
        grid=grid,
        in_specs=[
            pl.BlockSpec((RBLK, 128), lambda i: (i, 0)),
            pl.BlockSpec((1, 128), lambda i: (0, 0)),
            pl.BlockSpec((1, 128), lambda i: (0, 0)),
        ],
        out_specs=pl.BlockSpec((GBLK, 128), lambda i: (i, 0)),
        out_shape=jax.ShapeDtypeStruct((GROUPS, 128), jnp.float32),
    )(y, a, d)


def _bn_coeffs(st, g, be):
    mean = st[0] / float(ROWS)
    var = st[1] / float(ROWS) - mean * mean
    a = g / jnp.sqrt(var + 1e-5)
    d = be - mean * a
    return a[None, :], d[None, :]


def kernel(xyz, points, W0, b0, g0, be0, W1, b1, g1, be1, W2, b2, g2, be2):
    far0 = jax.random.randint(jax.random.key(7), (B,), 0, N).astype(jnp.int32)
    nxt_tmp = _fps(xyz, far0.reshape(B, 1))
    nxt = jnp.transpose(nxt_tmp, (1, 0, 2))
    new_xyz = jnp.transpose(nxt, (0, 2, 1))

    idx = _ballq(nxt, xyz)
    idx_flat = idx.reshape(ROWS)

    xyz_rows = jnp.transpose(xyz, (0, 2, 1)).reshape(B * N, 3)
    pts_rows = jnp.transpose(points, (0, 2, 1)).reshape(B * N, D)
    table = jnp.concatenate(
        [xyz_rows, pts_rows, jnp.zeros((B * N, CPAD - 3 - D), jnp.float32)], axis=1)
    gathered = _sc_gather(table, idx_flat)

    w0p = jnp.concatenate(
        [W0.T, jnp.zeros((CPAD - 3 - D, W0.shape[0]), jnp.float32)], axis=0)
    y1, st1 = _mlp1(gathered, nxt.reshape(GROUPS, 3), w0p, b0[None, :])
    a1, d1 = _bn_coeffs(st1, g0, be0)
    y2, st2 = _mlp_layer(y1, a1, d1, W1.T, b1[None, :])
    a2, d2 = _bn_coeffs(st2, g1, be1)
    y3, st3 = _mlp_layer(y2, a2, d2, W2.T, b2[None, :])
    a3, d3 = _bn_coeffs(st3, g2, be2)
    pooled = _pool(y3, a3, d3)
    new_points = jnp.transpose(pooled.reshape(B, S, 128), (0, 2, 1))
    return (new_xyz, new_points)
